# baseline (device time: 60439 ns/iter reference)
import jax
import jax.numpy as jnp
from jax import lax
from jax.experimental import pallas as pl
from jax.experimental.pallas import tpu as pltpu

M_BLOCK = 1024
D = 1024


def kernel(partial, gamma):
    p2d = partial.reshape(2 * M_BLOCK, D)
    g2d = gamma.reshape(1, D)

    def body(p_ref, g_ref, out_ref, send_sem, recv_sem):
        my_x = lax.axis_index("x")
        my_y = lax.axis_index("y")
        my_z = lax.axis_index("z")
        peer_x = 1 - my_x

        rdma = pltpu.make_async_remote_copy(
            src_ref=p_ref.at[pl.ds(peer_x * M_BLOCK, M_BLOCK), :],
            dst_ref=out_ref,
            send_sem=send_sem,
            recv_sem=recv_sem,
            device_id=(peer_x, my_y, my_z),
            device_id_type=pl.DeviceIdType.MESH,
        )
        rdma.start()
        rdma.wait()

        local = p_ref[pl.ds(my_x * M_BLOCK, M_BLOCK), :]
        y = out_ref[:, :] + local
        ms = jnp.mean(y * y, axis=-1, keepdims=True)
        out_ref[:, :] = y * lax.rsqrt(ms + 1e-6) * g_ref[0, :]

    return pl.pallas_call(
        body,
        out_shape=jax.ShapeDtypeStruct((M_BLOCK, D), jnp.float32),
        in_specs=[
            pl.BlockSpec(memory_space=pltpu.VMEM),
            pl.BlockSpec(memory_space=pltpu.VMEM),
        ],
        out_specs=pl.BlockSpec(memory_space=pltpu.VMEM),
        scratch_shapes=[
            pltpu.SemaphoreType.DMA,
            pltpu.SemaphoreType.DMA,
        ],
    )(p2d, g2d)


# device time: 37798 ns/iter; 1.5990x vs baseline; 1.5990x over previous
import jax
import jax.numpy as jnp
from jax import lax
from jax.experimental import pallas as pl
from jax.experimental.pallas import tpu as pltpu

M_BLOCK = 1024
D = 1024
CK = 128


def kernel(partial, gamma):
    p2d = partial.reshape(2 * M_BLOCK, D)
    g2d = gamma.reshape(1, D)

    def body(p_ref, g_ref, out_ref, comm_ref, send_sems, recv_sems):
        my_x = lax.axis_index("x")
        my_y = lax.axis_index("y")
        my_z = lax.axis_index("z")
        px = (1 - my_x, my_y, my_z)
        py = (my_x, 1 - my_y, my_z)
        pz = (my_x, my_y, 1 - my_z)
        me = (my_x, my_y, my_z)

        off_my = my_z * 512 + my_y * 256
        off_y = my_z * 512 + (1 - my_y) * 256
        off_z = (1 - my_z) * 512 + my_y * 256
        off_dg = (1 - my_z) * 512 + (1 - my_y) * 256

        barrier = pltpu.get_barrier_semaphore()
        for nbr in (px, py, pz):
            pl.semaphore_signal(
                barrier, inc=1, device_id=nbr,
                device_id_type=pl.DeviceIdType.MESH,
            )
        pl.semaphore_wait(barrier, 3)

        sends = []

        def send(src, dst_off, sem_i, slot_i, target):
            r = pltpu.make_async_remote_copy(
                src_ref=src,
                dst_ref=comm_ref.at[pl.ds(dst_off, CK), :],
                send_sem=send_sems.at[sem_i],
                recv_sem=recv_sems.at[slot_i],
                device_id=target,
                device_id_type=pl.DeviceIdType.MESH,
            )
            r.start()
            sends.append(r)

        def wait_slot(slot_i, off):
            r = pltpu.make_async_remote_copy(
                src_ref=comm_ref.at[pl.ds(off, CK), :],
                dst_ref=comm_ref.at[pl.ds(off, CK), :],
                send_sem=send_sems.at[slot_i],
                recv_sem=recv_sems.at[slot_i],
                device_id=me,
                device_id_type=pl.DeviceIdType.MESH,
            )
            r.wait_recv()

        def compute(off):
            y = comm_ref[pl.ds(off, CK), :] + p_ref[
                pl.ds(my_x * M_BLOCK + off, CK), :
            ]
            ms = jnp.mean(y * y, axis=-1, keepdims=True)
            out_ref[pl.ds(off, CK), :] = y * lax.rsqrt(ms + 1e-6) * g_ref[0, :]

        for c in range(2):
            send(
                p_ref.at[pl.ds((1 - my_x) * M_BLOCK + off_my + c * CK, CK), :],
                off_my + c * CK, sem_i=c, slot_i=c, target=px,
            )

        wait_slot(0, off_my)
        send(comm_ref.at[pl.ds(off_my, CK), :], off_my, 2, 2, py)
        send(comm_ref.at[pl.ds(off_my, CK), :], off_my, 3, 5, pz)
        compute(off_my)
        wait_slot(1, off_my + CK)
        send(comm_ref.at[pl.ds(off_my + CK, CK), :], off_my + CK, 4, 3, py)
        send(comm_ref.at[pl.ds(off_my + CK, CK), :], off_my + CK, 5, 6, pz)
        compute(off_my + CK)

        wait_slot(2, off_y)
        send(comm_ref.at[pl.ds(off_y, CK), :], off_y, 6, 7, pz)
        compute(off_y)
        wait_slot(3, off_y + CK)
        compute(off_y + CK)

        wait_slot(5, off_z)
        compute(off_z)
        wait_slot(6, off_z + CK)
        send(comm_ref.at[pl.ds(off_z + CK, CK), :], off_z + CK, 7, 4, py)
        compute(off_z + CK)

        wait_slot(7, off_dg)
        compute(off_dg)
        wait_slot(4, off_dg + CK)
        compute(off_dg + CK)

        for r in sends:
            r.wait_send()

    return pl.pallas_call(
        body,
        out_shape=jax.ShapeDtypeStruct((M_BLOCK, D), jnp.float32),
        in_specs=[
            pl.BlockSpec(memory_space=pltpu.VMEM),
            pl.BlockSpec(memory_space=pltpu.VMEM),
        ],
        out_specs=pl.BlockSpec(memory_space=pltpu.VMEM),
        scratch_shapes=[
            pltpu.VMEM((M_BLOCK, D), jnp.float32),
            pltpu.SemaphoreType.DMA((8,)),
            pltpu.SemaphoreType.DMA((8,)),
        ],
        compiler_params=pltpu.CompilerParams(collective_id=0),
    )(p2d, g2d)


# device time: 26026 ns/iter; 2.3223x vs baseline; 1.4523x over previous
import jax
import jax.numpy as jnp
from jax import lax
from jax.experimental import pallas as pl
from jax.experimental.pallas import tpu as pltpu

M_BLOCK = 1024
D = 1024
Q = 256
CK = 32
NC = Q // CK
NSEM = 3 * NC + 3


def kernel(partial, gamma):
    def body(p_ref, g_ref, out_ref, comm_ref, stage_ref, ostage_ref,
             send_sems, recv_sems, copy_sems):
        my_x = lax.axis_index("x")
        my_y = lax.axis_index("y")
        my_z = lax.axis_index("z")
        px = (1 - my_x, my_y, my_z)
        py = (my_x, 1 - my_y, my_z)
        pz = (my_x, my_y, 1 - my_z)
        me = (my_x, my_y, my_z)

        off_my = my_z * 512 + my_y * 256
        off_y = my_z * 512 + (1 - my_y) * 256
        off_z = (1 - my_z) * 512 + my_y * 256
        off_dg = (1 - my_z) * 512 + (1 - my_y) * 256

        X_Q, X_DG = 0, NC
        Y_Q, Y_FWD = NC + 1, 2 * NC + 1
        Z_Q, Z_FWD = 2 * NC + 2, 3 * NC + 2

        barrier = pltpu.get_barrier_semaphore()
        for nbr in (px, py, pz):
            pl.semaphore_signal(
                barrier, inc=1, device_id=nbr,
                device_id_type=pl.DeviceIdType.MESH,
            )
        pl.semaphore_wait(barrier, 3)

        sends = []

        def send(src, dst_off, rows, sem_i, slot_i, target):
            r = pltpu.make_async_remote_copy(
                src_ref=src,
                dst_ref=comm_ref.at[pl.ds(dst_off, rows), :],
                send_sem=send_sems.at[sem_i],
                recv_sem=recv_sems.at[slot_i],
                device_id=target,
                device_id_type=pl.DeviceIdType.MESH,
            )
            r.start()
            sends.append(r)

        def wait_slot(slot_i, off, rows):
            r = pltpu.make_async_remote_copy(
                src_ref=comm_ref.at[pl.ds(off, rows), :],
                dst_ref=comm_ref.at[pl.ds(off, rows), :],
                send_sem=send_sems.at[slot_i],
                recv_sem=recv_sems.at[slot_i],
                device_id=me,
                device_id_type=pl.DeviceIdType.MESH,
            )
            r.wait_recv()

        out_copies = []

        def compute(off, rows):
            y = comm_ref[pl.ds(off, rows), :].astype(jnp.float32) + p_ref[
                0, pl.ds(my_x * M_BLOCK + off, rows), :
            ]
            ms = jnp.mean(y * y, axis=-1, keepdims=True)
            ostage_ref[pl.ds(off, rows), :] = (
                y * lax.rsqrt(ms + 1e-6) * g_ref[:]
            )
            c = pltpu.make_async_copy(
                ostage_ref.at[pl.ds(off, rows), :],
                out_ref.at[pl.ds(off, rows), :],
                copy_sems.at[len(out_copies)],
            )
            c.start()
            out_copies.append(c)

        pblk = (1 - my_x) * M_BLOCK

        for c in range(NC):
            stage_ref[pl.ds(c * CK, CK), :] = p_ref[
                0, pl.ds(pblk + off_my + c * CK, CK), :
            ].astype(jnp.bfloat16)
            send(stage_ref.at[pl.ds(c * CK, CK), :],
                 off_my + c * CK, CK, sem_i=X_Q + c, slot_i=X_Q + c,
                 target=px)
        stage_ref[pl.ds(Q, 96), :] = p_ref[
            0, pl.ds(pblk + off_dg, 96), :
        ].astype(jnp.bfloat16)
        send(stage_ref.at[pl.ds(Q, 96), :],
             off_dg, 96, sem_i=X_DG, slot_i=X_DG, target=px)

        for c in range(NC):
            wait_slot(X_Q + c, off_my + c * CK, CK)
            send(comm_ref.at[pl.ds(off_my + c * CK, CK), :],
                 off_my + c * CK, CK, sem_i=Y_Q + c, slot_i=Y_Q + c,
                 target=py)
            send(comm_ref.at[pl.ds(off_my + c * CK, CK), :],
                 off_my + c * CK, CK, sem_i=Z_Q + c, slot_i=Z_Q + c,
                 target=pz)
            compute(off_my + c * CK, CK)

        fz_idx = (176 + CK - 1) // CK - 1
        for c in range(NC):
            wait_slot(Y_Q + c, off_y + c * CK, CK)
            if c == NC - 1:
                send(comm_ref.at[pl.ds(off_y + 176, 80), :],
                     off_y + 176, 80, sem_i=Z_FWD, slot_i=Z_FWD, target=pz)
            compute(off_y + c * CK, CK)
            wait_slot(Z_Q + c, off_z + c * CK, CK)
            if c == fz_idx:
                send(comm_ref.at[pl.ds(off_z + 96, 80), :],
                     off_z + 96, 80, sem_i=Y_FWD, slot_i=Y_FWD, target=py)
            compute(off_z + c * CK, CK)

        wait_slot(X_DG, off_dg, 96)
        compute(off_dg, 96)
        wait_slot(Y_FWD, off_dg + 96, 80)
        compute(off_dg + 96, 80)
        wait_slot(Z_FWD, off_dg + 176, 80)
        compute(off_dg + 176, 80)

        for r in sends:
            r.wait_send()
        for c in out_copies:
            c.wait()

    return pl.pallas_call(
        body,
        out_shape=jax.ShapeDtypeStruct((M_BLOCK, D), jnp.float32),
        in_specs=[
            pl.BlockSpec(memory_space=pltpu.VMEM),
            pl.BlockSpec(memory_space=pltpu.VMEM),
        ],
        out_specs=pl.BlockSpec(memory_space=pl.ANY),
        scratch_shapes=[
            pltpu.VMEM((M_BLOCK, D), jnp.bfloat16),
            pltpu.VMEM((Q + 96, D), jnp.bfloat16),
            pltpu.VMEM((M_BLOCK, D), jnp.float32),
            pltpu.SemaphoreType.DMA((NSEM,)),
            pltpu.SemaphoreType.DMA((NSEM,)),
            pltpu.SemaphoreType.DMA((3 * NC + 3,)),
        ],
        compiler_params=pltpu.CompilerParams(collective_id=0),
    )(partial, gamma)
